# baseline (speedup 1.0000x reference)
import functools

import jax
import jax.numpy as jnp
from jax import lax
from jax.experimental import pallas as pl
from jax.experimental.pallas import tpu as pltpu
from jax.experimental.pallas import tpu_sc as plsc

VOCAB = 100000
EMB = 16
BATCH = 1024


_NC = 2
_NS = 16
_NW = _NC * _NS
_B_PER_W = BATCH // _NW


def _sc_gather_body(table_hbm, idx_hbm, out_hbm, idx_v, rows_v, sem):
    wid = lax.axis_index("s") * _NC + lax.axis_index("c")
    base = wid * _B_PER_W
    pltpu.sync_copy(idx_hbm.at[pl.ds(base, _B_PER_W)], idx_v)
    pltpu.async_copy(table_hbm.at[idx_v], rows_v, sem).wait()
    pltpu.sync_copy(rows_v, out_hbm.at[pl.ds(base, _B_PER_W)])


@functools.cache
def _sc_gather_kernel():
    return pl.kernel(
        _sc_gather_body,
        out_type=jax.ShapeDtypeStruct((BATCH, EMB), jnp.float32),
        mesh=plsc.VectorSubcoreMesh(core_axis_name="c", subcore_axis_name="s"),
        scratch_types=[
            pltpu.VMEM((_B_PER_W,), jnp.int32),
            pltpu.VMEM((_B_PER_W, EMB), jnp.float32),
            pltpu.SemaphoreType.DMA,
        ],
        compiler_params=pltpu.CompilerParams(use_tc_tiling_on_sc=False),
    )


_TV = 2000
_KA = EMB + 1


def _proj_body(w_ref, x_ref, o_ref):
    o_ref[...] = lax.dot_general(
        w_ref[...],
        x_ref[...],
        (((1,), (1,)), ((), ())),
        preferred_element_type=jnp.float32,
    )


def _tc_project_t(w_aug, x_aug):
    return pl.pallas_call(
        _proj_body,
        grid=(VOCAB // _TV,),
        in_specs=[
            pl.BlockSpec((_TV, _KA), lambda j: (j, 0)),
            pl.BlockSpec((BATCH, _KA), lambda j: (0, 0)),
        ],
        out_specs=pl.BlockSpec((_TV, BATCH), lambda j: (j, 0)),
        out_shape=jax.ShapeDtypeStruct((VOCAB, BATCH), jnp.float32),
    )(w_aug, x_aug)


@jax.jit
def kernel(context_ids, embedding_weight, linear_weight, linear_bias):
    ids = context_ids.astype(jnp.int32)
    embeds = _sc_gather_kernel()(embedding_weight, ids)
    w_aug = jnp.concatenate([linear_weight, linear_bias[:, None]], axis=1)
    x_aug = jnp.concatenate([embeds, jnp.ones((BATCH, 1), jnp.float32)], axis=1)
    out_t = _tc_project_t(w_aug, x_aug)
    return out_t.T


# speedup vs baseline: 1.0197x; 1.0197x over previous
import functools

import jax
import jax.numpy as jnp
from jax import lax
from jax.experimental import pallas as pl
from jax.experimental.pallas import tpu as pltpu
from jax.experimental.pallas import tpu_sc as plsc

VOCAB = 100000
EMB = 16
BATCH = 1024


_NC = 2
_NS = 16
_NW = _NC * _NS
_B_PER_W = BATCH // _NW


def _sc_gather_body(table_hbm, idx_hbm, out_hbm, idx_v, rows_v, sem):
    wid = lax.axis_index("s") * _NC + lax.axis_index("c")
    base = wid * _B_PER_W
    pltpu.sync_copy(idx_hbm.at[pl.ds(base, _B_PER_W)], idx_v)
    pltpu.async_copy(table_hbm.at[idx_v], rows_v, sem).wait()
    pltpu.sync_copy(rows_v, out_hbm.at[pl.ds(base, _B_PER_W)])


@functools.cache
def _sc_gather_kernel():
    return pl.kernel(
        _sc_gather_body,
        out_type=jax.ShapeDtypeStruct((BATCH, EMB), jnp.float32),
        mesh=plsc.VectorSubcoreMesh(core_axis_name="c", subcore_axis_name="s"),
        scratch_types=[
            pltpu.VMEM((_B_PER_W,), jnp.int32),
            pltpu.VMEM((_B_PER_W, EMB), jnp.float32),
            pltpu.SemaphoreType.DMA,
        ],
        compiler_params=pltpu.CompilerParams(use_tc_tiling_on_sc=False),
    )


_TV = 5000
_KA = EMB + 1


def _proj_body(w_ref, x_ref, o_ref):
    o_ref[...] = lax.dot_general(
        w_ref[...],
        x_ref[...],
        (((1,), (1,)), ((), ())),
        preferred_element_type=jnp.float32,
    )


def _tc_project_t(w_aug, x_aug):
    return pl.pallas_call(
        _proj_body,
        grid=(VOCAB // _TV,),
        in_specs=[
            pl.BlockSpec((_TV, _KA), lambda j: (j, 0)),
            pl.BlockSpec((BATCH, _KA), lambda j: (0, 0)),
        ],
        out_specs=pl.BlockSpec((_TV, BATCH), lambda j: (j, 0)),
        out_shape=jax.ShapeDtypeStruct((VOCAB, BATCH), jnp.float32),
    )(w_aug, x_aug)


@jax.jit
def kernel(context_ids, embedding_weight, linear_weight, linear_bias):
    ids = context_ids.astype(jnp.int32)
    embeds = _sc_gather_kernel()(embedding_weight, ids)
    w_aug = jnp.concatenate([linear_weight, linear_bias[:, None]], axis=1)
    x_aug = jnp.concatenate([embeds, jnp.ones((BATCH, 1), jnp.float32)], axis=1)
    out_t = _tc_project_t(w_aug, x_aug)
    return out_t.T
